# baseline (speedup 1.0000x reference)
import functools

import jax
import jax.numpy as jnp
from jax.experimental import pallas as pl
from jax.experimental.pallas import tpu as pltpu

_SQRT2 = 1.4142135623730951

_dot = functools.partial(jnp.dot, preferred_element_type=jnp.float32)
_f32 = jnp.float32
_bf16 = jnp.bfloat16


def _gelu(x):
    return 0.5 * x * (1.0 + jax.lax.erf(x / _SQRT2))


def _ln_rows(x, g, b, ones_col):
    m = _dot(x, ones_col)
    ms = _dot(x * x, ones_col)
    v = ms - m * m
    return (x - m) * jax.lax.rsqrt(v + 1e-5) * g + b


def _ln0(xT, g, b, ones8_row):
    m = _dot(ones8_row, xT)[:1]
    ms = _dot(ones8_row, xT * xT)[:1]
    v = ms - m * m
    return (xT - m) * jax.lax.rsqrt(v + 1e-5) * g + b


def _cells_next(cprev, am, vm):
    c2 = cprev + vm * am.T
    nrm = jnp.maximum(jnp.sqrt(jnp.sum(c2 * c2, axis=0, keepdims=True)), 1.0)
    return c2 / nrm


def _colmv(w, c):
    return jnp.sum(w * c.T, axis=-1, keepdims=True)


def _tail(hn, j, whn_ref, wab_ref, wvb_ref, o832b_ref, aacc_ref, vacc_ref):
    phn = _dot(whn_ref[...], hn.astype(_bf16))
    ea = jnp.exp(phn[:32] + wab_ref[...])
    sa = _dot(o832b_ref[...], ea.astype(_bf16))[:1]
    wa = ea * (1.0 / sa)
    wv = jnp.tanh(phn[32:] + wvb_ref[...])

    @pl.when(j == 0)
    def _():
        aacc_ref[...] = jnp.zeros_like(aacc_ref)
        vacc_ref[...] = jnp.zeros_like(vacc_ref)

    aacc_ref[...] += jnp.sum(wa, axis=1, keepdims=True)
    vacc_ref[...] += jnp.sum(wv, axis=1, keepdims=True)


def _enc_kernel(x_ref, w1t_ref, b1_ref, g1_ref, be1_ref,
                w2t_ref, b2_ref, g2_ref, be2_ref,
                whn_ref, wab_ref, wvb_ref,
                o256_ref, o64_ref, o832b_ref,
                zt_ref, aacc_ref, vacc_ref):
    j = pl.program_id(0)
    h1 = _dot(x_ref[...], w1t_ref[...]) + b1_ref[...]
    h1 = _ln_rows(_gelu(h1), g1_ref[...], be1_ref[...], o256_ref[...])
    z = _ln_rows(_dot(h1.astype(_bf16), w2t_ref[...]) + b2_ref[...],
                 g2_ref[...], be2_ref[...], o64_ref[...])
    zt = z.astype(_bf16).T
    zt_ref[...] = zt
    _tail(zt, j, whn_ref, wab_ref, wvb_ref, o832b_ref,
          aacc_ref, vacc_ref)


def _make_step1_kernel(inv_b):
    def _step1_kernel(z_ref, a0_ref, v0_ref, rab_ref,
                      mmr_ref, mmb_ref, mng_ref, mnb_ref,
                      wz_ref, wrzm_ref, brz_ref, wnm_ref, bn_ref,
                      hng_ref, hnb_ref,
                      whn_ref, wab_ref, wvb_ref,
                      o864_ref, o832b_ref,
                      h_ref, aacc_ref, vacc_ref, cm_ref):
        j = pl.program_id(0)
        a0 = a0_ref[...] * inv_b
        v0 = v0_ref[...] * inv_b
        cellsT = v0 * a0.T
        cm_ref[...] = cellsT
        zb = z_ref[...]
        e = jnp.exp(rab_ref[...])
        ra = e / jnp.sum(e, axis=0, keepdims=True)
        rd = _colmv(cellsT, ra)
        mem0 = jnp.tanh(_colmv(mmr_ref[...], rd) + mmb_ref[...])
        mu = jnp.mean(mem0, axis=0, keepdims=True)
        var = jnp.mean(mem0 * mem0, axis=0, keepdims=True) - mu * mu
        mem = (mem0 - mu) * jax.lax.rsqrt(var + 1e-5) * mng_ref[...] + mnb_ref[...]
        pz = _dot(wz_ref[...], zb)
        cmc = _colmv(wrzm_ref[...], mem) + brz_ref[...]
        gz = jax.nn.sigmoid(pz[64:128] + cmc[64:])
        n = jnp.tanh(pz[128:] + (_colmv(wnm_ref[...], mem) + bn_ref[...]))
        hn = _ln0(gz * n, hng_ref[...], hnb_ref[...], o864_ref[...])
        h_ref[...] = hn.astype(_bf16)
        _tail(hn, j, whn_ref, wab_ref, wvb_ref, o832b_ref, aacc_ref, vacc_ref)
    return _step1_kernel


def _step_core(z_ref, h_ref, cellsT, rab_ref,
               wh_ref, wz_ref, wm_ref, wnh_ref, mmr_ref,
               mmb_ref, mng_ref, mnb_ref, brz_ref, bn_ref,
               hng_ref, hnb_ref, o864_ref, o832b_ref):
    hb = h_ref[...]
    h32 = hb.astype(_f32)
    ph = _dot(wh_ref[...], hb)
    e = jnp.exp(ph[:32] + rab_ref[...])
    eb = e.astype(_bf16)
    s = _dot(o832b_ref[...], eb)[:1]
    rd = _dot(cellsT.astype(_bf16), eb) * (1.0 / s)
    mem = _ln0(jnp.tanh(ph[32:96] + _dot(mmr_ref[...], rd.astype(_bf16))
                        + mmb_ref[...]),
               mng_ref[...], mnb_ref[...], o864_ref)
    pz = _dot(wz_ref[...], z_ref[...])
    pm = _dot(wm_ref[...], mem.astype(_bf16))
    pre = pz[:128] + pm[:128] + ph[96:224] + brz_ref[...]
    r = jax.nn.sigmoid(pre[:64])
    gz = jax.nn.sigmoid(pre[64:])
    rh = (r * h32).astype(_bf16)
    n = jnp.tanh(pz[128:] + pm[128:] + _dot(wnh_ref[...], rh) + bn_ref[...])
    return _ln0((1.0 - gz) * h32 + gz * n, hng_ref[...], hnb_ref[...],
                o864_ref)


def _make_step_kernel(inv_b):
    def _step_kernel(z_ref, h_ref, cp_ref, ap_ref, vp_ref, rab_ref,
                     wh_ref, wz_ref, wm_ref, wnh_ref, mmr_ref,
                     mmb_ref, mng_ref, mnb_ref, brz_ref, bn_ref,
                     hng_ref, hnb_ref,
                     whn_ref, wab_ref, wvb_ref,
                     o864_ref, o832b_ref,
                     ho_ref, aacc_ref, vacc_ref, cm_ref):
        j = pl.program_id(0)
        am = ap_ref[...] * inv_b
        vm = vp_ref[...] * inv_b
        cellsT = _cells_next(cp_ref[...], am, vm)
        cm_ref[...] = cellsT
        hn = _step_core(z_ref, h_ref, cellsT, rab_ref,
                        wh_ref, wz_ref, wm_ref, wnh_ref, mmr_ref,
                        mmb_ref, mng_ref, mnb_ref, brz_ref, bn_ref,
                        hng_ref, hnb_ref, o864_ref[...], o832b_ref)
        ho_ref[...] = hn.astype(_bf16)
        _tail(hn, j, whn_ref, wab_ref, wvb_ref, o832b_ref, aacc_ref, vacc_ref)
    return _step_kernel


def _make_final_kernel(inv_b):
    def _final_kernel(z_ref, h_ref, cp_ref, ap_ref, vp_ref, rab_ref,
                      wh_ref, wz_ref, wm_ref, wnh_ref, mmr_ref,
                      mmb_ref, mng_ref, mnb_ref, brz_ref, bn_ref,
                      hng_ref, hnb_ref,
                      dw1_ref, db1_ref, dg_ref, dbeta_ref,
                      dw2_ref, db2_ref,
                      o864_ref, o832b_ref,
                      out_ref):
        am = ap_ref[...] * inv_b
        vm = vp_ref[...] * inv_b
        cellsT = _cells_next(cp_ref[...], am, vm)
        hn = _step_core(z_ref, h_ref, cellsT, rab_ref,
                        wh_ref, wz_ref, wm_ref, wnh_ref, mmr_ref,
                        mmb_ref, mng_ref, mnb_ref, brz_ref, bn_ref,
                        hng_ref, hnb_ref, o864_ref[...], o832b_ref)
        d = _ln0(_gelu(_dot(dw1_ref[...], hn.astype(_bf16)) + db1_ref[...]),
                 dg_ref[...], dbeta_ref[...], o864_ref[...])
        out_ref[...] = _dot(dw2_ref[...], d.astype(_bf16)) + db2_ref[...]
    return _final_kernel


def _full(shape):
    return pl.BlockSpec(shape, lambda j: tuple(0 for _ in shape))


def _batch_spec(f, blk):
    return pl.BlockSpec((f, blk), lambda j: (0, j))


_PARAMS = pltpu.CompilerParams(
    dimension_semantics=("arbitrary",),
)


def kernel(x, enc_w1, enc_b1, enc_g1, enc_beta1, enc_w2, enc_b2, enc_g2,
           enc_beta2, ra_w, ra_b, wa_w, wa_b, wv_w, wv_b, mm_w, mm_b, mn_g,
           mn_beta, wr_w, wr_b, wz_w, wz_b, wn_w, wn_b, hn_g, hn_beta,
           dec_w1, dec_b1, dec_g, dec_beta, dec_w2, dec_b2,
           interpret=False):
    B, _ = x.shape
    inv_b = 1.0 / B

    def tern(w):
        return jnp.sign(w) * (jnp.abs(w) > 0.1).astype(w.dtype)

    raq = tern(ra_w)
    waq = tern(wa_w)
    wvq = tern(wv_w)

    col = lambda v: v[:, None].astype(_f32)
    row = lambda v: v[None, :].astype(_f32)
    b16 = lambda a: a.astype(_bf16)

    mmh, mmr = mm_w[:, :64], mm_w[:, 64:]
    wrzz = jnp.concatenate([wr_w[:, :64], wz_w[:, :64]], axis=0)
    wrzm = jnp.concatenate([wr_w[:, 64:128], wz_w[:, 64:128]], axis=0)
    wrzh = jnp.concatenate([wr_w[:, 128:], wz_w[:, 128:]], axis=0)
    brz = jnp.concatenate([wr_b, wz_b], axis=0)[:, None]
    wnz, wnm, wnh = wn_w[:, :64], wn_w[:, 64:128], wn_w[:, 128:]

    wh = b16(jnp.concatenate([raq, mmh, wrzh], axis=0))
    wzg = b16(jnp.concatenate([wrzz, wnz], axis=0))
    wmg = b16(jnp.concatenate([wrzm, wnm], axis=0))
    whn = b16(jnp.concatenate([waq, wvq], axis=0))
    wnh16 = b16(wnh)

    rab, wab, wvb = col(ra_b), col(wa_b), col(wv_b)
    mmb, mng, mnb = col(mm_b), col(mn_g), col(mn_beta)
    bn, hng, hnb = col(wn_b), col(hn_g), col(hn_beta)
    db1, dg, dbeta, db2 = col(dec_b1), col(dec_g), col(dec_beta), col(dec_b2)

    o256 = jnp.full((256, 1), 1.0 / 256, _f32)
    o64 = jnp.full((64, 1), 1.0 / 64, _f32)
    o864 = jnp.full((8, 64), 1.0 / 64, _f32)
    o832b = jnp.ones((8, 32), _bf16)

    def sizes(pref):
        blk = pref
        while B % blk:
            blk //= 2
        return blk, B // blk

    eblk, eg = sizes(2048)
    blk, g = sizes(8192)

    zt, a0, v0 = pl.pallas_call(
        _enc_kernel,
        grid=(eg,),
        in_specs=[
            pl.BlockSpec((eblk, 768), lambda j: (j, 0)),
            _full((768, 256)), _full((1, 256)), _full((1, 256)), _full((1, 256)),
            _full((256, 64)), _full((1, 64)), _full((1, 64)), _full((1, 64)),
            _full((96, 64)), _full((32, 1)), _full((64, 1)),
            _full((256, 1)), _full((64, 1)), _full((8, 32)),
        ],
        out_specs=[
            _batch_spec(64, eblk),
            _full((32, 1)), _full((64, 1)),
        ],
        out_shape=[
            jax.ShapeDtypeStruct((64, B), _bf16),
            jax.ShapeDtypeStruct((32, 1), _f32),
            jax.ShapeDtypeStruct((64, 1), _f32),
        ],
        compiler_params=_PARAMS,
        name="rc_encoder",
        interpret=interpret,
    )(b16(x), b16(enc_w1.T), row(enc_b1), row(enc_g1), row(enc_beta1),
      b16(enc_w2.T), row(enc_b2), row(enc_g2), row(enc_beta2),
      whn, wab, wvb, o256, o64, o832b)

    step_outs = [
        jax.ShapeDtypeStruct((64, B), _bf16),
        jax.ShapeDtypeStruct((32, 1), _f32),
        jax.ShapeDtypeStruct((64, 1), _f32),
        jax.ShapeDtypeStruct((64, 32), _f32),
    ]
    step_out_specs = [_batch_spec(64, blk), _full((32, 1)), _full((64, 1)),
                      _full((64, 32))]
    zb = _batch_spec(64, blk)

    h, ap, vp, cm = pl.pallas_call(
        _make_step1_kernel(inv_b),
        grid=(g,),
        in_specs=[
            zb, _full((32, 1)), _full((64, 1)), _full((32, 1)),
            _full((64, 64)), _full((64, 1)), _full((64, 1)), _full((64, 1)),
            _full((192, 64)), _full((128, 64)), _full((128, 1)),
            _full((64, 64)), _full((64, 1)),
            _full((64, 1)), _full((64, 1)),
            _full((96, 64)), _full((32, 1)), _full((64, 1)),
            _full((8, 64)), _full((8, 32)),
        ],
        out_specs=step_out_specs,
        out_shape=step_outs,
        compiler_params=_PARAMS,
        name="rc_step1",
        interpret=interpret,
    )(zt, a0, v0, rab, mmr, mmb, mng, mnb, wzg, wrzm, brz, wnm, bn,
      hng, hnb, whn, wab, wvb, o864, o832b)

    mid = pl.pallas_call(
        _make_step_kernel(inv_b),
        grid=(g,),
        in_specs=[
            zb, zb, _full((64, 32)), _full((32, 1)), _full((64, 1)),
            _full((32, 1)),
            _full((224, 64)), _full((192, 64)), _full((192, 64)),
            _full((64, 64)), _full((64, 64)),
            _full((64, 1)), _full((64, 1)), _full((64, 1)),
            _full((128, 1)), _full((64, 1)),
            _full((64, 1)), _full((64, 1)),
            _full((96, 64)), _full((32, 1)), _full((64, 1)),
            _full((8, 64)), _full((8, 32)),
        ],
        out_specs=step_out_specs,
        out_shape=step_outs,
        compiler_params=_PARAMS,
        name="rc_step",
        interpret=interpret,
    )
    for _ in range(6):
        h, ap, vp, cm = mid(zt, h, cm, ap, vp, rab, wh, wzg, wmg, wnh16,
                            b16(mmr), mmb, mng, mnb, brz, bn, hng, hnb,
                            whn, wab, wvb, o864, o832b)

    outT = pl.pallas_call(
        _make_final_kernel(inv_b),
        grid=(g,),
        in_specs=[
            zb, zb, _full((64, 32)), _full((32, 1)), _full((64, 1)),
            _full((32, 1)),
            _full((224, 64)), _full((192, 64)), _full((192, 64)),
            _full((64, 64)), _full((64, 64)),
            _full((64, 1)), _full((64, 1)), _full((64, 1)),
            _full((128, 1)), _full((64, 1)),
            _full((64, 1)), _full((64, 1)),
            _full((64, 64)), _full((64, 1)), _full((64, 1)), _full((64, 1)),
            _full((2, 64)), _full((2, 1)),
            _full((8, 64)), _full((8, 32)),
        ],
        out_specs=pl.BlockSpec((2, blk), lambda j: (0, j)),
        out_shape=jax.ShapeDtypeStruct((2, B), _f32),
        compiler_params=_PARAMS,
        name="rc_final",
        interpret=interpret,
    )(zt, h, cm, ap, vp, rab, wh, wzg, wmg, wnh16, b16(mmr),
      mmb, mng, mnb, brz, bn, hng, hnb,
      b16(dec_w1), db1, dg, dbeta, b16(dec_w2), db2, o864, o832b)

    return outT.T


# speedup vs baseline: 1.2585x; 1.2585x over previous
import functools

import jax
import jax.numpy as jnp
from jax.experimental import pallas as pl
from jax.experimental.pallas import tpu as pltpu

_SQRT2 = 1.4142135623730951

_dot = functools.partial(jnp.dot, preferred_element_type=jnp.float32)
_f32 = jnp.float32
_bf16 = jnp.bfloat16


def _gelu(x):
    return 0.5 * x * (1.0 + jax.lax.erf(x / _SQRT2))


def _ln_rows(x, g, b, ones_col):
    m = _dot(x, ones_col)
    ms = _dot(x * x, ones_col)
    v = ms - m * m
    return (x - m) * jax.lax.rsqrt(v + 1e-5) * g + b


def _ln0(xT, g, b, ones8_row):
    m = _dot(ones8_row, xT)[:1]
    ms = _dot(ones8_row, xT * xT)[:1]
    v = ms - m * m
    return (xT - m) * jax.lax.rsqrt(v + 1e-5) * g + b


def _cells_next(cprev, am, vm):
    c2 = cprev + vm * am.T
    nrm = jnp.maximum(jnp.sqrt(jnp.sum(c2 * c2, axis=0, keepdims=True)), 1.0)
    return c2 / nrm


def _colmv(w, c):
    return jnp.sum(w * c.T, axis=-1, keepdims=True)


def _tail(hn16, j, whn_ref, wab_ref, wvb_ref, o832b_ref, ocol_ref,
          aacc_ref, vacc_ref):
    phn = _dot(whn_ref[...], hn16)
    ea = jnp.exp(phn[:32] + wab_ref[...])
    sa = _dot(o832b_ref[...], ea.astype(_bf16))[:1]
    wv = jnp.tanh(phn[32:] + wvb_ref[...])
    rec = (1.0 / sa).T

    @pl.when(j == 0)
    def _():
        aacc_ref[...] = jnp.zeros_like(aacc_ref)
        vacc_ref[...] = jnp.zeros_like(vacc_ref)

    aacc_ref[...] += _dot(ea, rec)
    vacc_ref[...] += _dot(wv, ocol_ref[...])


def _enc_kernel(x_ref, w1t_ref, b1_ref, g1_ref, be1_ref,
                w2t_ref, b2_ref, g2_ref, be2_ref,
                whn_ref, wab_ref, wvb_ref,
                o256_ref, o64_ref, o832b_ref, oecol_ref,
                zt_ref, aacc_ref, vacc_ref):
    j = pl.program_id(0)
    h1 = _dot(x_ref[...].astype(_bf16), w1t_ref[...]) + b1_ref[...]
    h1 = _ln_rows(_gelu(h1), g1_ref[...], be1_ref[...], o256_ref[...])
    z = _ln_rows(_dot(h1.astype(_bf16), w2t_ref[...]) + b2_ref[...],
                 g2_ref[...], be2_ref[...], o64_ref[...])
    zt = z.astype(_bf16).T
    zt_ref[...] = zt
    _tail(zt, j, whn_ref, wab_ref, wvb_ref, o832b_ref, oecol_ref,
          aacc_ref, vacc_ref)


def _make_step1_kernel(inv_b):
    def _step1_kernel(z_ref, a0_ref, v0_ref, rab_ref,
                      mmr_ref, mmb_ref, mng_ref, mnb_ref,
                      wz_ref, wrzm_ref, brz_ref, wnm_ref, bn_ref,
                      hng_ref, hnb_ref,
                      whn_ref, wab_ref, wvb_ref,
                      o864_ref, o832b_ref, ocol_ref,
                      h_ref, aacc_ref, vacc_ref, cm_ref):
        j = pl.program_id(0)
        a0 = a0_ref[...] * inv_b
        v0 = v0_ref[...] * inv_b
        cellsT = v0 * a0.T
        cm_ref[...] = cellsT
        zb = z_ref[...]
        e = jnp.exp(rab_ref[...])
        ra = e / jnp.sum(e, axis=0, keepdims=True)
        rd = _colmv(cellsT, ra)
        mem0 = jnp.tanh(_colmv(mmr_ref[...], rd) + mmb_ref[...])
        mu = jnp.mean(mem0, axis=0, keepdims=True)
        var = jnp.mean(mem0 * mem0, axis=0, keepdims=True) - mu * mu
        mem = (mem0 - mu) * jax.lax.rsqrt(var + 1e-5) * mng_ref[...] + mnb_ref[...]
        pz = _dot(wz_ref[...], zb)
        cmc = _colmv(wrzm_ref[...], mem) + brz_ref[...]
        gz = jax.nn.sigmoid(pz[64:128] + cmc[64:])
        n = jnp.tanh(pz[128:] + (_colmv(wnm_ref[...], mem) + bn_ref[...]))
        hn = _ln0(gz * n, hng_ref[...], hnb_ref[...], o864_ref[...])
        h_ref[...] = hn
        _tail(hn.astype(_bf16), j, whn_ref, wab_ref, wvb_ref, o832b_ref,
              ocol_ref, aacc_ref, vacc_ref)
    return _step1_kernel


def _step_core(z_ref, h_ref, cellsT, rab_ref,
               whmm_ref, wrzn_ref, wnh_ref, mmr_ref,
               mmb_ref, mng_ref, mnb_ref, brz_ref, bn_ref,
               hng_ref, hnb_ref, o864_ref, o832b_ref):
    h32 = h_ref[...]
    h16 = h32.astype(_bf16)
    ph = _dot(whmm_ref[...], h16)
    e = jnp.exp(ph[:32] + rab_ref[...])
    eb = e.astype(_bf16)
    s = _dot(o832b_ref[...], eb)[:1]
    rd = _dot(cellsT.astype(_bf16), eb) * (1.0 / s)
    mem = _ln0(jnp.tanh(ph[32:] + _dot(mmr_ref[...], rd.astype(_bf16))
                        + mmb_ref[...]),
               mng_ref[...], mnb_ref[...], o864_ref)
    xfull = jnp.concatenate([z_ref[...], mem.astype(_bf16), h16], axis=0)
    pg = _dot(wrzn_ref[...], xfull)
    pre = pg[:128] + brz_ref[...]
    r = jax.nn.sigmoid(pre[:64])
    gz = jax.nn.sigmoid(pre[64:])
    rh = (r * h32).astype(_bf16)
    n = jnp.tanh(pg[128:] + _dot(wnh_ref[...], rh) + bn_ref[...])
    return _ln0(h32 + gz * (n - h32), hng_ref[...], hnb_ref[...], o864_ref)


def _make_step_kernel(inv_b):
    def _step_kernel(z_ref, h_ref, cp_ref, ap_ref, vp_ref, rab_ref,
                     whmm_ref, wrzn_ref, wnh_ref, mmr_ref,
                     mmb_ref, mng_ref, mnb_ref, brz_ref, bn_ref,
                     hng_ref, hnb_ref,
                     whn_ref, wab_ref, wvb_ref,
                     o864_ref, o832b_ref, ocol_ref,
                     ho_ref, aacc_ref, vacc_ref, cm_ref):
        j = pl.program_id(0)
        am = ap_ref[...] * inv_b
        vm = vp_ref[...] * inv_b
        cellsT = _cells_next(cp_ref[...], am, vm)
        cm_ref[...] = cellsT
        hn = _step_core(z_ref, h_ref, cellsT, rab_ref,
                        whmm_ref, wrzn_ref, wnh_ref, mmr_ref,
                        mmb_ref, mng_ref, mnb_ref, brz_ref, bn_ref,
                        hng_ref, hnb_ref, o864_ref[...], o832b_ref)
        ho_ref[...] = hn
        _tail(hn.astype(_bf16), j, whn_ref, wab_ref, wvb_ref, o832b_ref,
              ocol_ref, aacc_ref, vacc_ref)
    return _step_kernel


def _make_final_kernel(inv_b):
    def _final_kernel(z_ref, h_ref, cp_ref, ap_ref, vp_ref, rab_ref,
                      whmm_ref, wrzn_ref, wnh_ref, mmr_ref,
                      mmb_ref, mng_ref, mnb_ref, brz_ref, bn_ref,
                      hng_ref, hnb_ref,
                      dw1_ref, db1_ref, dg_ref, dbeta_ref,
                      dw2_ref, db2_ref,
                      o864_ref, o832b_ref,
                      out_ref):
        am = ap_ref[...] * inv_b
        vm = vp_ref[...] * inv_b
        cellsT = _cells_next(cp_ref[...], am, vm)
        hn = _step_core(z_ref, h_ref, cellsT, rab_ref,
                        whmm_ref, wrzn_ref, wnh_ref, mmr_ref,
                        mmb_ref, mng_ref, mnb_ref, brz_ref, bn_ref,
                        hng_ref, hnb_ref, o864_ref[...], o832b_ref)
        d = _ln0(_gelu(_dot(dw1_ref[...], hn.astype(_bf16)) + db1_ref[...]),
                 dg_ref[...], dbeta_ref[...], o864_ref[...])
        out_ref[...] = _dot(dw2_ref[...], d.astype(_bf16)) + db2_ref[...]
    return _final_kernel


def _full(shape):
    return pl.BlockSpec(shape, lambda j: tuple(0 for _ in shape))


def _batch_spec(f, blk):
    return pl.BlockSpec((f, blk), lambda j: (0, j))


_PARAMS = pltpu.CompilerParams(
    dimension_semantics=("arbitrary",),
)


def kernel(x, enc_w1, enc_b1, enc_g1, enc_beta1, enc_w2, enc_b2, enc_g2,
           enc_beta2, ra_w, ra_b, wa_w, wa_b, wv_w, wv_b, mm_w, mm_b, mn_g,
           mn_beta, wr_w, wr_b, wz_w, wz_b, wn_w, wn_b, hn_g, hn_beta,
           dec_w1, dec_b1, dec_g, dec_beta, dec_w2, dec_b2,
           interpret=False):
    B, _ = x.shape
    inv_b = 1.0 / B

    def tern(w):
        return jnp.sign(w) * (jnp.abs(w) > 0.1).astype(w.dtype)

    raq = tern(ra_w)
    waq = tern(wa_w)
    wvq = tern(wv_w)

    col = lambda v: v[:, None].astype(_f32)
    row = lambda v: v[None, :].astype(_f32)
    b16 = lambda a: a.astype(_bf16)

    mmh, mmr = mm_w[:, :64], mm_w[:, 64:]
    wrzz = jnp.concatenate([wr_w[:, :64], wz_w[:, :64]], axis=0)
    wrzm = jnp.concatenate([wr_w[:, 64:128], wz_w[:, 64:128]], axis=0)
    wrzh = jnp.concatenate([wr_w[:, 128:], wz_w[:, 128:]], axis=0)
    brz = jnp.concatenate([wr_b, wz_b], axis=0)[:, None]
    wnz, wnm, wnh = wn_w[:, :64], wn_w[:, 64:128], wn_w[:, 128:]

    whmm = b16(jnp.concatenate([raq, mmh], axis=0))
    z64 = jnp.zeros((64, 64), _f32)
    wrzn = b16(jnp.concatenate([
        jnp.concatenate([wrzz, wrzm, wrzh], axis=1),
        jnp.concatenate([wnz, wnm, z64], axis=1),
    ], axis=0))
    wzg = b16(jnp.concatenate([wrzz, wnz], axis=0))
    whn = b16(jnp.concatenate([waq, wvq], axis=0))
    wnh16 = b16(wnh)

    rab, wab, wvb = col(ra_b), col(wa_b), col(wv_b)
    mmb, mng, mnb = col(mm_b), col(mn_g), col(mn_beta)
    bn, hng, hnb = col(wn_b), col(hn_g), col(hn_beta)
    db1, dg, dbeta, db2 = col(dec_b1), col(dec_g), col(dec_beta), col(dec_b2)

    o256 = jnp.full((256, 1), 1.0 / 256, _f32)
    o64 = jnp.full((64, 1), 1.0 / 64, _f32)
    o864 = jnp.full((8, 64), 1.0 / 64, _f32)
    o832b = jnp.ones((8, 32), _bf16)

    def sizes(pref):
        blk = pref
        while B % blk:
            blk //= 2
        return blk, B // blk

    eblk, eg = sizes(2048)
    blk, g = sizes(8192)
    oecol = jnp.ones((eblk, 1), _f32)
    ocol = jnp.ones((blk, 1), _f32)

    zt, a0, v0 = pl.pallas_call(
        _enc_kernel,
        grid=(eg,),
        in_specs=[
            pl.BlockSpec((eblk, 768), lambda j: (j, 0)),
            _full((768, 256)), _full((1, 256)), _full((1, 256)), _full((1, 256)),
            _full((256, 64)), _full((1, 64)), _full((1, 64)), _full((1, 64)),
            _full((96, 64)), _full((32, 1)), _full((64, 1)),
            _full((256, 1)), _full((64, 1)), _full((8, 32)), _full((eblk, 1)),
        ],
        out_specs=[
            _batch_spec(64, eblk),
            _full((32, 1)), _full((64, 1)),
        ],
        out_shape=[
            jax.ShapeDtypeStruct((64, B), _bf16),
            jax.ShapeDtypeStruct((32, 1), _f32),
            jax.ShapeDtypeStruct((64, 1), _f32),
        ],
        compiler_params=_PARAMS,
        name="rc_encoder",
        interpret=interpret,
    )(x, b16(enc_w1.T), row(enc_b1), row(enc_g1), row(enc_beta1),
      b16(enc_w2.T), row(enc_b2), row(enc_g2), row(enc_beta2),
      whn, wab, wvb, o256, o64, o832b, oecol)

    step_outs = [
        jax.ShapeDtypeStruct((64, B), _f32),
        jax.ShapeDtypeStruct((32, 1), _f32),
        jax.ShapeDtypeStruct((64, 1), _f32),
        jax.ShapeDtypeStruct((64, 32), _f32),
    ]
    step_out_specs = [_batch_spec(64, blk), _full((32, 1)), _full((64, 1)),
                      _full((64, 32))]
    zb = _batch_spec(64, blk)

    h, ap, vp, cm = pl.pallas_call(
        _make_step1_kernel(inv_b),
        grid=(g,),
        in_specs=[
            zb, _full((32, 1)), _full((64, 1)), _full((32, 1)),
            _full((64, 64)), _full((64, 1)), _full((64, 1)), _full((64, 1)),
            _full((192, 64)), _full((128, 64)), _full((128, 1)),
            _full((64, 64)), _full((64, 1)),
            _full((64, 1)), _full((64, 1)),
            _full((96, 64)), _full((32, 1)), _full((64, 1)),
            _full((8, 64)), _full((8, 32)), _full((blk, 1)),
        ],
        out_specs=step_out_specs,
        out_shape=step_outs,
        compiler_params=_PARAMS,
        name="rc_step1",
        interpret=interpret,
    )(zt, a0, v0, rab, mmr, mmb, mng, mnb, wzg, wrzm, brz, wnm, bn,
      hng, hnb, whn, wab, wvb, o864, o832b, ocol)

    mid = pl.pallas_call(
        _make_step_kernel(inv_b),
        grid=(g,),
        in_specs=[
            zb, zb, _full((64, 32)), _full((32, 1)), _full((64, 1)),
            _full((32, 1)),
            _full((96, 64)), _full((192, 192)), _full((64, 64)), _full((64, 64)),
            _full((64, 1)), _full((64, 1)), _full((64, 1)),
            _full((128, 1)), _full((64, 1)),
            _full((64, 1)), _full((64, 1)),
            _full((96, 64)), _full((32, 1)), _full((64, 1)),
            _full((8, 64)), _full((8, 32)), _full((blk, 1)),
        ],
        out_specs=step_out_specs,
        out_shape=step_outs,
        compiler_params=_PARAMS,
        name="rc_step",
        interpret=interpret,
    )
    for _ in range(6):
        h, ap, vp, cm = mid(zt, h, cm, ap, vp, rab, whmm, wrzn, wnh16,
                            b16(mmr), mmb, mng, mnb, brz, bn, hng, hnb,
                            whn, wab, wvb, o864, o832b, ocol)

    outT = pl.pallas_call(
        _make_final_kernel(inv_b),
        grid=(g,),
        in_specs=[
            zb, zb, _full((64, 32)), _full((32, 1)), _full((64, 1)),
            _full((32, 1)),
            _full((96, 64)), _full((192, 192)), _full((64, 64)), _full((64, 64)),
            _full((64, 1)), _full((64, 1)), _full((64, 1)),
            _full((128, 1)), _full((64, 1)),
            _full((64, 1)), _full((64, 1)),
            _full((64, 64)), _full((64, 1)), _full((64, 1)), _full((64, 1)),
            _full((2, 64)), _full((2, 1)),
            _full((8, 64)), _full((8, 32)),
        ],
        out_specs=pl.BlockSpec((2, blk), lambda j: (0, j)),
        out_shape=jax.ShapeDtypeStruct((2, B), _f32),
        compiler_params=_PARAMS,
        name="rc_final",
        interpret=interpret,
    )(zt, h, cm, ap, vp, rab, whmm, wrzn, wnh16, b16(mmr),
      mmb, mng, mnb, brz, bn, hng, hnb,
      b16(dec_w1), db1, dg, dbeta, b16(dec_w2), db2, o864, o832b)

    return outT.T


# speedup vs baseline: 1.3731x; 1.0910x over previous
import functools

import jax
import jax.numpy as jnp
from jax.experimental import pallas as pl
from jax.experimental.pallas import tpu as pltpu

_SQRT2 = 1.4142135623730951

_dot = functools.partial(jnp.dot, preferred_element_type=jnp.float32)
_f32 = jnp.float32
_bf16 = jnp.bfloat16


def _gelu(x):
    return 0.5 * x * (1.0 + jax.lax.erf(x / _SQRT2))


def _ln_rows(x, g, b, ones_col):
    m = _dot(x, ones_col)
    ms = _dot(x * x, ones_col)
    v = ms - m * m
    return (x - m) * jax.lax.rsqrt(v + 1e-5) * g + b


def _ln_rows_raw(x, ones_col):
    m = _dot(x, ones_col)
    ms = _dot(x * x, ones_col)
    v = ms - m * m
    return (x - m) * jax.lax.rsqrt(v + 1e-5)


def _ln0(xT, g, b, ones8_row):
    m = _dot(ones8_row, xT)[:1]
    ms = _dot(ones8_row, xT * xT)[:1]
    v = ms - m * m
    return (xT - m) * jax.lax.rsqrt(v + 1e-5) * g + b


def _cells_next(cprev, am, vm):
    c2 = cprev + vm * am.T
    nrm = jnp.maximum(jnp.sqrt(jnp.sum(c2 * c2, axis=0, keepdims=True)), 1.0)
    return c2 / nrm


def _colmv(w, c):
    return jnp.sum(w * c.T, axis=-1, keepdims=True)


def _wa_wv(hne, whn_ref, o832b_ref, ocol_ref):
    phn = _dot(whn_ref[...], hne)
    ea = jnp.exp(phn[:32])
    sa = _dot(o832b_ref[...], ea.astype(_bf16))[:1]
    wv = jnp.tanh(phn[32:])
    rec = (1.0 / sa).T
    return _dot(ea, rec), _dot(wv, ocol_ref[...])


def _with_ones(a16):
    ones8 = jnp.ones((8, a16.shape[1]), _bf16)
    return jnp.concatenate([a16, ones8], axis=0)


def _enc_kernel(x_ref, w1t_ref, b1_ref, w2tg_ref, b2e_ref, g2_ref, be2_ref,
                whn_ref, o256_ref, o64_ref, o832b_ref, oecol_ref,
                zt_ref, aacc_ref, vacc_ref):
    j = pl.program_id(0)
    y = _gelu(_dot(x_ref[...].astype(_bf16), w1t_ref[...]) + b1_ref[...])
    h1n = _ln_rows_raw(y, o256_ref[...])
    z = _ln_rows(_dot(h1n.astype(_bf16), w2tg_ref[...]) + b2e_ref[...],
                 g2_ref[...], be2_ref[...], o64_ref[...])
    zt = z.astype(_bf16).T
    zt_ref[...] = zt
    wa_s, wv_s = _wa_wv(_with_ones(zt), whn_ref, o832b_ref, oecol_ref)

    @pl.when(j == 0)
    def _():
        aacc_ref[...] = jnp.zeros_like(aacc_ref)
        vacc_ref[...] = jnp.zeros_like(vacc_ref)

    aacc_ref[...] += wa_s
    vacc_ref[...] += wv_s


def _make_step1_kernel(inv_b):
    def _step1_kernel(z_ref, a0_ref, v0_ref, rab_ref,
                      mmr_ref, mmb_ref, mng_ref, mnb_ref,
                      wz_ref, wrzm_ref, brz_ref, wnm_ref, bn_ref,
                      hng_ref, hnb_ref,
                      whn_ref, o864_ref, o832b_ref, ocol_ref,
                      h_ref, aacc_ref, vacc_ref, cm_ref):
        j = pl.program_id(0)
        a0 = a0_ref[...] * inv_b
        v0 = v0_ref[...] * inv_b
        cellsT = v0 * a0.T
        cm_ref[...] = cellsT
        zb = z_ref[...]
        e = jnp.exp(rab_ref[...])
        ra = e / jnp.sum(e, axis=0, keepdims=True)
        rd = _colmv(cellsT, ra)
        mem0 = jnp.tanh(_colmv(mmr_ref[...], rd) + mmb_ref[...])
        mu = jnp.mean(mem0, axis=0, keepdims=True)
        var = jnp.mean(mem0 * mem0, axis=0, keepdims=True) - mu * mu
        mem = (mem0 - mu) * jax.lax.rsqrt(var + 1e-5) * mng_ref[...] + mnb_ref[...]
        pz = _dot(wz_ref[...], zb)
        cmc = _colmv(wrzm_ref[...], mem) + brz_ref[...]
        gz = jax.nn.sigmoid(pz[64:128] + cmc[64:])
        n = jnp.tanh(pz[128:] + (_colmv(wnm_ref[...], mem) + bn_ref[...]))
        hn = _ln0(gz * n, hng_ref[...], hnb_ref[...], o864_ref[...])
        h_ref[...] = hn
        wa_s, wv_s = _wa_wv(_with_ones(hn.astype(_bf16)), whn_ref,
                            o832b_ref, ocol_ref)

        @pl.when(j == 0)
        def _():
            aacc_ref[...] = jnp.zeros_like(aacc_ref)
            vacc_ref[...] = jnp.zeros_like(vacc_ref)

        aacc_ref[...] += wa_s
        vacc_ref[...] += wv_s
    return _step1_kernel


def _step_core(z_ref, h_ref, cellsT,
               whmm_ref, wrzn_ref, wnh_ref, mmr_ref,
               mng_ref, mnb_ref, hng_ref, hnb_ref, o864_ref, o832b_ref):
    h32 = h_ref[...]
    he = _with_ones(h32.astype(_bf16))
    ph = _dot(whmm_ref[...], he)
    e = jnp.exp(ph[:32])
    eb = e.astype(_bf16)
    s = _dot(o832b_ref[...], eb)[:1]
    rd = _dot(cellsT.astype(_bf16), eb) * (1.0 / s)
    mem = _ln0(jnp.tanh(ph[32:] + _dot(mmr_ref[...], rd.astype(_bf16))),
               mng_ref[...], mnb_ref[...], o864_ref)
    xfull = jnp.concatenate([z_ref[...], mem.astype(_bf16), he], axis=0)
    pg = _dot(wrzn_ref[...], xfull)
    r = jax.nn.sigmoid(pg[:64])
    gz = jax.nn.sigmoid(pg[64:128])
    rh = (r * h32).astype(_bf16)
    n = jnp.tanh(pg[128:] + _dot(wnh_ref[...], rh))
    return _ln0(h32 + gz * (n - h32), hng_ref[...], hnb_ref[...], o864_ref)


def _make_mid_kernel(inv_b):
    def _mid_kernel(z_ref, h_ref, cm0_ref, a0_ref, v0_ref,
                    whmm_ref, wrzn_ref, wnh_ref, mmr_ref,
                    mng_ref, mnb_ref, hng_ref, hnb_ref,
                    whn_ref, o864_ref, o832b_ref, ocol_ref,
                    ho_ref, ap6_ref, vp6_ref, cm6_ref):
        s = pl.program_id(0)
        j = pl.program_id(1)
        idx = jnp.maximum(s - 1, 0)
        first = s == 0
        am = jnp.where(first, a0_ref[...], ap6_ref[idx]) * inv_b
        vm = jnp.where(first, v0_ref[...], vp6_ref[idx]) * inv_b
        cprev = jnp.where(first, cm0_ref[...], cm6_ref[idx])
        cellsT = _cells_next(cprev, am, vm)
        cm6_ref[s] = cellsT
        hn = _step_core(z_ref, h_ref, cellsT,
                        whmm_ref, wrzn_ref, wnh_ref, mmr_ref,
                        mng_ref, mnb_ref, hng_ref, hnb_ref,
                        o864_ref[...], o832b_ref)
        ho_ref[...] = hn
        wa_s, wv_s = _wa_wv(_with_ones(hn.astype(_bf16)), whn_ref,
                            o832b_ref, ocol_ref)

        @pl.when(j == 0)
        def _():
            ap6_ref[s] = jnp.zeros_like(wa_s)
            vp6_ref[s] = jnp.zeros_like(wv_s)

        ap6_ref[s] += wa_s
        vp6_ref[s] += wv_s
    return _mid_kernel


def _make_final_kernel(inv_b):
    def _final_kernel(z_ref, h_ref, cm6_ref, ap6_ref, vp6_ref,
                      whmm_ref, wrzn_ref, wnh_ref, mmr_ref,
                      mng_ref, mnb_ref, hng_ref, hnb_ref,
                      dw1_ref, db1_ref, dg_ref, dbeta_ref,
                      dw2_ref, db2_ref,
                      o864_ref, o832b_ref,
                      out_ref):
        am = ap6_ref[5] * inv_b
        vm = vp6_ref[5] * inv_b
        cellsT = _cells_next(cm6_ref[5], am, vm)
        hn = _step_core(z_ref, h_ref, cellsT,
                        whmm_ref, wrzn_ref, wnh_ref, mmr_ref,
                        mng_ref, mnb_ref, hng_ref, hnb_ref,
                        o864_ref[...], o832b_ref)
        d = _ln0(_gelu(_dot(dw1_ref[...], hn.astype(_bf16)) + db1_ref[...]),
                 dg_ref[...], dbeta_ref[...], o864_ref[...])
        out_ref[...] = _dot(dw2_ref[...], d.astype(_bf16)) + db2_ref[...]
    return _final_kernel


def _full(shape):
    return pl.BlockSpec(shape, lambda j: tuple(0 for _ in shape))


def _full2(shape):
    return pl.BlockSpec(shape, lambda s, j: tuple(0 for _ in shape))


def _batch_spec(f, blk):
    return pl.BlockSpec((f, blk), lambda j: (0, j))


_PARAMS = pltpu.CompilerParams(
    dimension_semantics=("arbitrary",),
)
_PARAMS2 = pltpu.CompilerParams(
    dimension_semantics=("arbitrary", "arbitrary"),
)


def kernel(x, enc_w1, enc_b1, enc_g1, enc_beta1, enc_w2, enc_b2, enc_g2,
           enc_beta2, ra_w, ra_b, wa_w, wa_b, wv_w, wv_b, mm_w, mm_b, mn_g,
           mn_beta, wr_w, wr_b, wz_w, wz_b, wn_w, wn_b, hn_g, hn_beta,
           dec_w1, dec_b1, dec_g, dec_beta, dec_w2, dec_b2,
           interpret=False):
    B, _ = x.shape
    inv_b = 1.0 / B

    def tern(w):
        return jnp.sign(w) * (jnp.abs(w) > 0.1).astype(w.dtype)

    raq = tern(ra_w)
    waq = tern(wa_w)
    wvq = tern(wv_w)

    col = lambda v: v[:, None].astype(_f32)
    row = lambda v: v[None, :].astype(_f32)
    b16 = lambda a: a.astype(_bf16)

    mmh, mmr = mm_w[:, :64], mm_w[:, 64:]
    wrzz = jnp.concatenate([wr_w[:, :64], wz_w[:, :64]], axis=0)
    wrzm = jnp.concatenate([wr_w[:, 64:128], wz_w[:, 64:128]], axis=0)
    wrzh = jnp.concatenate([wr_w[:, 128:], wz_w[:, 128:]], axis=0)
    brz = jnp.concatenate([wr_b, wz_b], axis=0)[:, None]
    wnz, wnm, wnh = wn_w[:, :64], wn_w[:, 64:128], wn_w[:, 128:]

    def bias_cols(bias_col):
        return jnp.concatenate(
            [bias_col, jnp.zeros((bias_col.shape[0], 7), _f32)], axis=1)

    whmm = b16(jnp.concatenate(
        [jnp.concatenate([raq, mmh], axis=0),
         bias_cols(jnp.concatenate([ra_b, mm_b], axis=0)[:, None])],
        axis=1))
    wrzn = b16(jnp.concatenate([
        jnp.concatenate(
            [jnp.concatenate([wrzz, wrzm, wrzh], axis=1),
             jnp.concatenate([wnz, wnm, jnp.zeros((64, 64), _f32)], axis=1)],
            axis=0),
        bias_cols(jnp.concatenate([wr_b, wz_b, wn_b], axis=0)[:, None]),
    ], axis=1))
    wzg = b16(jnp.concatenate([wrzz, wnz], axis=0))
    whn = b16(jnp.concatenate(
        [jnp.concatenate([waq, wvq], axis=0),
         bias_cols(jnp.concatenate([wa_b, wv_b], axis=0)[:, None])],
        axis=1))
    wnh16 = b16(wnh)

    w2tg = b16(enc_w2.T * enc_g1[:, None])
    b2eff = row(enc_b2 + enc_beta1 @ enc_w2.T)

    rab, mmb = col(ra_b), col(mm_b)
    mng, mnb = col(mn_g), col(mn_beta)
    bn, hng, hnb = col(wn_b), col(hn_g), col(hn_beta)
    db1, dg, dbeta, db2 = col(dec_b1), col(dec_g), col(dec_beta), col(dec_b2)

    o256 = jnp.full((256, 1), 1.0 / 256, _f32)
    o64 = jnp.full((64, 1), 1.0 / 64, _f32)
    o864 = jnp.full((8, 64), 1.0 / 64, _f32)
    o832b = jnp.ones((8, 32), _bf16)

    def sizes(pref):
        blk = pref
        while B % blk:
            blk //= 2
        return blk, B // blk

    eblk, eg = sizes(2048)
    blk, g = sizes(8192)
    oecol = jnp.ones((eblk, 1), _f32)
    ocol = jnp.ones((blk, 1), _f32)

    zt, a0, v0 = pl.pallas_call(
        _enc_kernel,
        grid=(eg,),
        in_specs=[
            pl.BlockSpec((eblk, 768), lambda j: (j, 0)),
            _full((768, 256)), _full((1, 256)),
            _full((256, 64)), _full((1, 64)), _full((1, 64)), _full((1, 64)),
            _full((96, 72)),
            _full((256, 1)), _full((64, 1)), _full((8, 32)), _full((eblk, 1)),
        ],
        out_specs=[
            _batch_spec(64, eblk),
            _full((32, 1)), _full((64, 1)),
        ],
        out_shape=[
            jax.ShapeDtypeStruct((64, B), _bf16),
            jax.ShapeDtypeStruct((32, 1), _f32),
            jax.ShapeDtypeStruct((64, 1), _f32),
        ],
        compiler_params=_PARAMS,
        name="rc_encoder",
        interpret=interpret,
    )(x, b16(enc_w1.T), row(enc_b1), w2tg, b2eff, row(enc_g2), row(enc_beta2),
      whn, o256, o64, o832b, oecol)

    zb = _batch_spec(64, blk)

    h, ap, vp, cm = pl.pallas_call(
        _make_step1_kernel(inv_b),
        grid=(g,),
        in_specs=[
            zb, _full((32, 1)), _full((64, 1)), _full((32, 1)),
            _full((64, 64)), _full((64, 1)), _full((64, 1)), _full((64, 1)),
            _full((192, 64)), _full((128, 64)), _full((128, 1)),
            _full((64, 64)), _full((64, 1)),
            _full((64, 1)), _full((64, 1)),
            _full((96, 72)), _full((8, 64)), _full((8, 32)), _full((blk, 1)),
        ],
        out_specs=[_batch_spec(64, blk), _full((32, 1)), _full((64, 1)),
                   _full((64, 32))],
        out_shape=[
            jax.ShapeDtypeStruct((64, B), _f32),
            jax.ShapeDtypeStruct((32, 1), _f32),
            jax.ShapeDtypeStruct((64, 1), _f32),
            jax.ShapeDtypeStruct((64, 32), _f32),
        ],
        compiler_params=_PARAMS,
        name="rc_step1",
        interpret=interpret,
    )(zt, a0, v0, rab, mmr, mmb, mng, mnb, wzg, wrzm, brz, wnm, bn,
      hng, hnb, whn, o864, o832b, ocol)

    h, ap6, vp6, cm6 = pl.pallas_call(
        _make_mid_kernel(inv_b),
        grid=(6, g),
        in_specs=[
            pl.BlockSpec((64, blk), lambda s, j: (0, j)),
            pl.BlockSpec((64, blk), lambda s, j: (0, j)),
            _full2((64, 32)), _full2((32, 1)), _full2((64, 1)),
            _full2((96, 72)), _full2((192, 200)), _full2((64, 64)),
            _full2((64, 64)),
            _full2((64, 1)), _full2((64, 1)), _full2((64, 1)), _full2((64, 1)),
            _full2((96, 72)), _full2((8, 64)), _full2((8, 32)),
            _full2((blk, 1)),
        ],
        out_specs=[
            pl.BlockSpec((64, blk), lambda s, j: (0, j)),
            _full2((6, 32, 1)), _full2((6, 64, 1)), _full2((6, 64, 32)),
        ],
        out_shape=[
            jax.ShapeDtypeStruct((64, B), _f32),
            jax.ShapeDtypeStruct((6, 32, 1), _f32),
            jax.ShapeDtypeStruct((6, 64, 1), _f32),
            jax.ShapeDtypeStruct((6, 64, 32), _f32),
        ],
        input_output_aliases={1: 0},
        compiler_params=_PARAMS2,
        name="rc_mid6",
        interpret=interpret,
    )(zt, h, cm, ap, vp, whmm, wrzn, wnh16, b16(mmr),
      mng, mnb, hng, hnb, whn, o864, o832b, ocol)

    outT = pl.pallas_call(
        _make_final_kernel(inv_b),
        grid=(g,),
        in_specs=[
            zb, _batch_spec(64, blk),
            _full((6, 64, 32)), _full((6, 32, 1)), _full((6, 64, 1)),
            _full((96, 72)), _full((192, 200)), _full((64, 64)),
            _full((64, 64)),
            _full((64, 1)), _full((64, 1)), _full((64, 1)), _full((64, 1)),
            _full((64, 64)), _full((64, 1)), _full((64, 1)), _full((64, 1)),
            _full((2, 64)), _full((2, 1)),
            _full((8, 64)), _full((8, 32)),
        ],
        out_specs=pl.BlockSpec((2, blk), lambda j: (0, j)),
        out_shape=jax.ShapeDtypeStruct((2, B), _f32),
        compiler_params=_PARAMS,
        name="rc_final",
        interpret=interpret,
    )(zt, h, cm6, ap6, vp6, whmm, wrzn, wnh16, b16(mmr),
      mng, mnb, hng, hnb,
      b16(dec_w1), db1, dg, dbeta, b16(dec_w2), db2, o864, o832b)

    return outT.T


# speedup vs baseline: 1.4111x; 1.0277x over previous
import functools

import jax
import jax.numpy as jnp
from jax.experimental import pallas as pl
from jax.experimental.pallas import tpu as pltpu

_SQRT2 = 1.4142135623730951

_dot = functools.partial(jnp.dot, preferred_element_type=jnp.float32)
_f32 = jnp.float32
_bf16 = jnp.bfloat16


def _gelu(x):
    return 0.5 * x * (1.0 + jax.lax.erf(x / _SQRT2))


def _ln_rows(x, g, b, ones_col):
    m = _dot(x, ones_col)
    ms = _dot(x * x, ones_col)
    v = ms - m * m
    return (x - m) * jax.lax.rsqrt(v + 1e-5) * g + b


def _ln_rows_raw(x, ones_col):
    m = _dot(x, ones_col)
    ms = _dot(x * x, ones_col)
    v = ms - m * m
    return (x - m) * jax.lax.rsqrt(v + 1e-5)


def _ln0(xT, g, b, ones8_row):
    m = _dot(ones8_row, xT)[:1]
    ms = _dot(ones8_row, xT * xT)[:1]
    v = ms - m * m
    return (xT - m) * jax.lax.rsqrt(v + 1e-5) * g + b


def _cells_next(cprev, am, vm):
    c2 = cprev + vm * am.T
    nrm = jnp.maximum(jnp.sqrt(jnp.sum(c2 * c2, axis=0, keepdims=True)), 1.0)
    return c2 / nrm


def _colmv(w, c):
    return jnp.sum(w * c.T, axis=-1, keepdims=True)


def _wa_wv(hne, whn_ref, o832b_ref, ocol_ref):
    phn = _dot(whn_ref[...], hne)
    ea = jnp.exp(phn[:32])
    sa = _dot(o832b_ref[...], ea.astype(_bf16))[:1]
    wv = jnp.tanh(phn[32:])
    rec = (1.0 / sa).T
    return _dot(ea, rec), _dot(wv, ocol_ref[...])


def _with_ones(a16):
    ones8 = jnp.ones((8, a16.shape[1]), _bf16)
    return jnp.concatenate([a16, ones8], axis=0)


def _enc_kernel(x_ref, w1t_ref, b1_ref, w2tg_ref, b2e_ref, g2_ref, be2_ref,
                whn_ref, o256_ref, o64_ref, o832b_ref, oecol_ref,
                zt_ref, aacc_ref, vacc_ref):
    j = pl.program_id(0)
    y = _gelu(_dot(x_ref[...].astype(_bf16), w1t_ref[...]) + b1_ref[...])
    h1n = _ln_rows_raw(y, o256_ref[...])
    z = _ln_rows(_dot(h1n.astype(_bf16), w2tg_ref[...]) + b2e_ref[...],
                 g2_ref[...], be2_ref[...], o64_ref[...])
    zt = z.astype(_bf16).T
    zt_ref[...] = zt
    wa_s, wv_s = _wa_wv(_with_ones(zt), whn_ref, o832b_ref, oecol_ref)

    @pl.when(j == 0)
    def _():
        aacc_ref[...] = jnp.zeros_like(aacc_ref)
        vacc_ref[...] = jnp.zeros_like(vacc_ref)

    aacc_ref[...] += wa_s
    vacc_ref[...] += wv_s


def _make_step1_kernel(inv_b):
    def _step1_kernel(z_ref, a0_ref, v0_ref, rab_ref,
                      mmr_ref, mmb_ref, mng_ref, mnb_ref,
                      wz_ref, wrzm_ref, brz_ref, wnm_ref, bn_ref,
                      hng_ref, hnb_ref,
                      whn_ref, o864_ref, o832b_ref, ocol_ref,
                      h_ref, aacc_ref, vacc_ref, cm_ref):
        j = pl.program_id(0)
        a0 = a0_ref[...] * inv_b
        v0 = v0_ref[...] * inv_b
        cellsT = v0 * a0.T
        cm_ref[...] = cellsT
        zb = z_ref[...]
        e = jnp.exp(rab_ref[...])
        ra = e / jnp.sum(e, axis=0, keepdims=True)
        rd = _colmv(cellsT, ra)
        mem0 = jnp.tanh(_colmv(mmr_ref[...], rd) + mmb_ref[...])
        mu = jnp.mean(mem0, axis=0, keepdims=True)
        var = jnp.mean(mem0 * mem0, axis=0, keepdims=True) - mu * mu
        mem = (mem0 - mu) * jax.lax.rsqrt(var + 1e-5) * mng_ref[...] + mnb_ref[...]
        pz = _dot(wz_ref[...], zb)
        cmc = _colmv(wrzm_ref[...], mem) + brz_ref[...]
        gz = jax.nn.sigmoid(pz[64:128] + cmc[64:])
        n = jnp.tanh(pz[128:] + (_colmv(wnm_ref[...], mem) + bn_ref[...]))
        hn = _ln0(gz * n, hng_ref[...], hnb_ref[...], o864_ref[...])
        h_ref[...] = hn
        wa_s, wv_s = _wa_wv(_with_ones(hn.astype(_bf16)), whn_ref,
                            o832b_ref, ocol_ref)

        @pl.when(j == 0)
        def _():
            aacc_ref[...] = jnp.zeros_like(aacc_ref)
            vacc_ref[...] = jnp.zeros_like(vacc_ref)

        aacc_ref[...] += wa_s
        vacc_ref[...] += wv_s
    return _step1_kernel


def _step_core(z_ref, h_ref, cellsT,
               whmm_ref, wrzn_ref, wnh_ref, mmr_ref,
               mng_ref, mnb_ref, hng_ref, hnb_ref, o864_ref, o832b_ref):
    h32 = h_ref[...]
    he = _with_ones(h32.astype(_bf16))
    ph = _dot(whmm_ref[...], he)
    e = jnp.exp(ph[:32])
    eb = e.astype(_bf16)
    s = _dot(o832b_ref[...], eb)[:1]
    rd = _dot(cellsT.astype(_bf16), eb) * (1.0 / s)
    mem = _ln0(jnp.tanh(ph[32:] + _dot(mmr_ref[...], rd.astype(_bf16))),
               mng_ref[...], mnb_ref[...], o864_ref)
    xfull = jnp.concatenate([z_ref[...], mem.astype(_bf16), he], axis=0)
    pg = _dot(wrzn_ref[...], xfull)
    r = jax.nn.sigmoid(pg[:64])
    gz = jax.nn.sigmoid(pg[64:128])
    rh = (r * h32).astype(_bf16)
    n = jnp.tanh(pg[128:] + _dot(wnh_ref[...], rh))
    return _ln0(h32 + gz * (n - h32), hng_ref[...], hnb_ref[...], o864_ref)


def _make_mid_kernel(inv_b):
    def _mid_kernel(z_ref, h_ref, cm0_ref, a0_ref, v0_ref,
                    whmm_ref, wrzn_ref, wnh_ref, mmr_ref,
                    mng_ref, mnb_ref, hng_ref, hnb_ref,
                    whn_ref, o864_ref, o832b_ref, ocol_ref,
                    ho_ref, ap6_ref, vp6_ref, cm6_ref):
        s = pl.program_id(0)
        j = pl.program_id(1)
        idx = jnp.maximum(s - 1, 0)
        first = s == 0
        am = jnp.where(first, a0_ref[...], ap6_ref[idx]) * inv_b
        vm = jnp.where(first, v0_ref[...], vp6_ref[idx]) * inv_b
        cprev = jnp.where(first, cm0_ref[...], cm6_ref[idx])
        cellsT = _cells_next(cprev, am, vm)
        cm6_ref[s] = cellsT
        hn = _step_core(z_ref, h_ref, cellsT,
                        whmm_ref, wrzn_ref, wnh_ref, mmr_ref,
                        mng_ref, mnb_ref, hng_ref, hnb_ref,
                        o864_ref[...], o832b_ref)
        ho_ref[...] = hn
        wa_s, wv_s = _wa_wv(_with_ones(hn.astype(_bf16)), whn_ref,
                            o832b_ref, ocol_ref)

        @pl.when(j == 0)
        def _():
            ap6_ref[s] = jnp.zeros_like(wa_s)
            vp6_ref[s] = jnp.zeros_like(wv_s)

        ap6_ref[s] += wa_s
        vp6_ref[s] += wv_s
    return _mid_kernel


def _make_final_kernel(inv_b):
    def _final_kernel(z_ref, h_ref, cm6_ref, ap6_ref, vp6_ref,
                      whmm_ref, wrzn_ref, wnh_ref, mmr_ref,
                      mng_ref, mnb_ref, hng_ref, hnb_ref,
                      dw1_ref, db1_ref, dg_ref, dbeta_ref,
                      dw2_ref, db2_ref,
                      o864_ref, o832b_ref,
                      out_ref):
        am = ap6_ref[5] * inv_b
        vm = vp6_ref[5] * inv_b
        cellsT = _cells_next(cm6_ref[5], am, vm)
        hn = _step_core(z_ref, h_ref, cellsT,
                        whmm_ref, wrzn_ref, wnh_ref, mmr_ref,
                        mng_ref, mnb_ref, hng_ref, hnb_ref,
                        o864_ref[...], o832b_ref)
        d = _ln0(_gelu(_dot(dw1_ref[...], hn.astype(_bf16)) + db1_ref[...]),
                 dg_ref[...], dbeta_ref[...], o864_ref[...])
        out_ref[...] = _dot(dw2_ref[...], d.astype(_bf16)) + db2_ref[...]
    return _final_kernel


def _full(shape):
    return pl.BlockSpec(shape, lambda j: tuple(0 for _ in shape))


def _full2(shape):
    return pl.BlockSpec(shape, lambda s, j: tuple(0 for _ in shape))


def _batch_spec(f, blk):
    return pl.BlockSpec((f, blk), lambda j: (0, j))


_PARAMS = pltpu.CompilerParams(
    dimension_semantics=("arbitrary",),
)
_PARAMS2 = pltpu.CompilerParams(
    dimension_semantics=("arbitrary", "arbitrary"),
)


def kernel(x, enc_w1, enc_b1, enc_g1, enc_beta1, enc_w2, enc_b2, enc_g2,
           enc_beta2, ra_w, ra_b, wa_w, wa_b, wv_w, wv_b, mm_w, mm_b, mn_g,
           mn_beta, wr_w, wr_b, wz_w, wz_b, wn_w, wn_b, hn_g, hn_beta,
           dec_w1, dec_b1, dec_g, dec_beta, dec_w2, dec_b2,
           interpret=False):
    B, _ = x.shape
    inv_b = 1.0 / B

    def tern(w):
        return jnp.sign(w) * (jnp.abs(w) > 0.1).astype(w.dtype)

    raq = tern(ra_w)
    waq = tern(wa_w)
    wvq = tern(wv_w)

    col = lambda v: v[:, None].astype(_f32)
    row = lambda v: v[None, :].astype(_f32)
    b16 = lambda a: a.astype(_bf16)

    mmh, mmr = mm_w[:, :64], mm_w[:, 64:]
    wrzz = jnp.concatenate([wr_w[:, :64], wz_w[:, :64]], axis=0)
    wrzm = jnp.concatenate([wr_w[:, 64:128], wz_w[:, 64:128]], axis=0)
    wrzh = jnp.concatenate([wr_w[:, 128:], wz_w[:, 128:]], axis=0)
    brz = jnp.concatenate([wr_b, wz_b], axis=0)[:, None]
    wnz, wnm, wnh = wn_w[:, :64], wn_w[:, 64:128], wn_w[:, 128:]

    def bias_cols(bias_col):
        return jnp.concatenate(
            [bias_col, jnp.zeros((bias_col.shape[0], 7), _f32)], axis=1)

    whmm = b16(jnp.concatenate(
        [jnp.concatenate([raq, mmh], axis=0),
         bias_cols(jnp.concatenate([ra_b, mm_b], axis=0)[:, None])],
        axis=1))
    wrzn = b16(jnp.concatenate([
        jnp.concatenate(
            [jnp.concatenate([wrzz, wrzm, wrzh], axis=1),
             jnp.concatenate([wnz, wnm, jnp.zeros((64, 64), _f32)], axis=1)],
            axis=0),
        bias_cols(jnp.concatenate([wr_b, wz_b, wn_b], axis=0)[:, None]),
    ], axis=1))
    wzg = b16(jnp.concatenate([wrzz, wnz], axis=0))
    whn = b16(jnp.concatenate(
        [jnp.concatenate([waq, wvq], axis=0),
         bias_cols(jnp.concatenate([wa_b, wv_b], axis=0)[:, None])],
        axis=1))
    wnh16 = b16(wnh)

    w2tg = b16(enc_w2.T * enc_g1[:, None])
    b2eff = row(enc_b2 + enc_beta1 @ enc_w2.T)

    rab, mmb = col(ra_b), col(mm_b)
    mng, mnb = col(mn_g), col(mn_beta)
    bn, hng, hnb = col(wn_b), col(hn_g), col(hn_beta)
    db1, dg, dbeta, db2 = col(dec_b1), col(dec_g), col(dec_beta), col(dec_b2)

    o256 = jnp.full((256, 1), 1.0 / 256, _f32)
    o64 = jnp.full((64, 1), 1.0 / 64, _f32)
    o864 = jnp.full((8, 64), 1.0 / 64, _f32)
    o832b = jnp.ones((8, 32), _bf16)

    def sizes(pref):
        blk = pref
        while B % blk:
            blk //= 2
        return blk, B // blk

    eblk, eg = sizes(4096)
    blk, g = sizes(8192)
    oecol = jnp.ones((eblk, 1), _f32)
    ocol = jnp.ones((blk, 1), _f32)

    zt, a0, v0 = pl.pallas_call(
        _enc_kernel,
        grid=(eg,),
        in_specs=[
            pl.BlockSpec((eblk, 768), lambda j: (j, 0)),
            _full((768, 256)), _full((1, 256)),
            _full((256, 64)), _full((1, 64)), _full((1, 64)), _full((1, 64)),
            _full((96, 72)),
            _full((256, 1)), _full((64, 1)), _full((8, 32)), _full((eblk, 1)),
        ],
        out_specs=[
            _batch_spec(64, eblk),
            _full((32, 1)), _full((64, 1)),
        ],
        out_shape=[
            jax.ShapeDtypeStruct((64, B), _bf16),
            jax.ShapeDtypeStruct((32, 1), _f32),
            jax.ShapeDtypeStruct((64, 1), _f32),
        ],
        compiler_params=_PARAMS,
        name="rc_encoder",
        interpret=interpret,
    )(x, b16(enc_w1.T), row(enc_b1), w2tg, b2eff, row(enc_g2), row(enc_beta2),
      whn, o256, o64, o832b, oecol)

    zb = _batch_spec(64, blk)

    h, ap, vp, cm = pl.pallas_call(
        _make_step1_kernel(inv_b),
        grid=(g,),
        in_specs=[
            zb, _full((32, 1)), _full((64, 1)), _full((32, 1)),
            _full((64, 64)), _full((64, 1)), _full((64, 1)), _full((64, 1)),
            _full((192, 64)), _full((128, 64)), _full((128, 1)),
            _full((64, 64)), _full((64, 1)),
            _full((64, 1)), _full((64, 1)),
            _full((96, 72)), _full((8, 64)), _full((8, 32)), _full((blk, 1)),
        ],
        out_specs=[_batch_spec(64, blk), _full((32, 1)), _full((64, 1)),
                   _full((64, 32))],
        out_shape=[
            jax.ShapeDtypeStruct((64, B), _f32),
            jax.ShapeDtypeStruct((32, 1), _f32),
            jax.ShapeDtypeStruct((64, 1), _f32),
            jax.ShapeDtypeStruct((64, 32), _f32),
        ],
        compiler_params=_PARAMS,
        name="rc_step1",
        interpret=interpret,
    )(zt, a0, v0, rab, mmr, mmb, mng, mnb, wzg, wrzm, brz, wnm, bn,
      hng, hnb, whn, o864, o832b, ocol)

    h, ap6, vp6, cm6 = pl.pallas_call(
        _make_mid_kernel(inv_b),
        grid=(6, g),
        in_specs=[
            pl.BlockSpec((64, blk), lambda s, j: (0, j)),
            pl.BlockSpec((64, blk), lambda s, j: (0, j)),
            _full2((64, 32)), _full2((32, 1)), _full2((64, 1)),
            _full2((96, 72)), _full2((192, 200)), _full2((64, 64)),
            _full2((64, 64)),
            _full2((64, 1)), _full2((64, 1)), _full2((64, 1)), _full2((64, 1)),
            _full2((96, 72)), _full2((8, 64)), _full2((8, 32)),
            _full2((blk, 1)),
        ],
        out_specs=[
            pl.BlockSpec((64, blk), lambda s, j: (0, j)),
            _full2((6, 32, 1)), _full2((6, 64, 1)), _full2((6, 64, 32)),
        ],
        out_shape=[
            jax.ShapeDtypeStruct((64, B), _f32),
            jax.ShapeDtypeStruct((6, 32, 1), _f32),
            jax.ShapeDtypeStruct((6, 64, 1), _f32),
            jax.ShapeDtypeStruct((6, 64, 32), _f32),
        ],
        input_output_aliases={1: 0},
        compiler_params=_PARAMS2,
        name="rc_mid6",
        interpret=interpret,
    )(zt, h, cm, ap, vp, whmm, wrzn, wnh16, b16(mmr),
      mng, mnb, hng, hnb, whn, o864, o832b, ocol)

    outT = pl.pallas_call(
        _make_final_kernel(inv_b),
        grid=(g,),
        in_specs=[
            zb, _batch_spec(64, blk),
            _full((6, 64, 32)), _full((6, 32, 1)), _full((6, 64, 1)),
            _full((96, 72)), _full((192, 200)), _full((64, 64)),
            _full((64, 64)),
            _full((64, 1)), _full((64, 1)), _full((64, 1)), _full((64, 1)),
            _full((64, 64)), _full((64, 1)), _full((64, 1)), _full((64, 1)),
            _full((2, 64)), _full((2, 1)),
            _full((8, 64)), _full((8, 32)),
        ],
        out_specs=pl.BlockSpec((2, blk), lambda j: (0, j)),
        out_shape=jax.ShapeDtypeStruct((2, B), _f32),
        compiler_params=_PARAMS,
        name="rc_final",
        interpret=interpret,
    )(zt, h, cm6, ap6, vp6, whmm, wrzn, wnh16, b16(mmr),
      mng, mnb, hng, hnb,
      b16(dec_w1), db1, dg, dbeta, b16(dec_w2), db2, o864, o832b)

    return outT.T
